# baseline (speedup 1.0000x reference)
import functools

import jax
import jax.numpy as jnp
from jax import lax
from jax.experimental import pallas as pl
from jax.experimental.pallas import tpu as pltpu
from jax.experimental.pallas import tpu_sc as plsc

_NC = 2
_NS = 16
_NW = _NC * _NS
_L = 16
_D = 1024
_BATCH = 4
_SEQ = 2048
_ROWS_W = _SEQ // _NW
_CH = 32
_PE_ELEMS = _ROWS_W * _D
_CH_ELEMS = _CH * _D
_CHUNKS_PER_BATCH = _ROWS_W // _CH


def _sc_body(x_hbm, pe_hbm, o_hbm, pe_v, x_v):
    wid = lax.axis_index("s") * _NC + lax.axis_index("c")
    pe_base = wid * _PE_ELEMS
    pltpu.sync_copy(pe_hbm.at[pl.ds(pe_base, _PE_ELEMS)], pe_v)

    def chunk_iter(t, carry):
        b = t // _CHUNKS_PER_BATCH
        c = t % _CHUNKS_PER_BATCH
        off = c * _CH_ELEMS
        base = b * (_SEQ * _D) + pe_base + off
        pltpu.sync_copy(x_hbm.at[pl.ds(base, _CH_ELEMS)], x_v)

        def vec_body(i, carry2):
            s = pl.ds(i * _L, _L)
            sp = pl.ds(off + i * _L, _L)
            x_v[s] = x_v[s] + pe_v[sp]
            return carry2

        lax.fori_loop(0, _CH_ELEMS // _L, vec_body, 0)
        pltpu.sync_copy(x_v, o_hbm.at[pl.ds(base, _CH_ELEMS)])
        return carry

    lax.fori_loop(0, _BATCH * _CHUNKS_PER_BATCH, chunk_iter, 0)


_sc_kernel = functools.partial(
    pl.kernel,
    out_type=jax.ShapeDtypeStruct((_BATCH * _SEQ * _D,), jnp.float32),
    mesh=plsc.VectorSubcoreMesh(core_axis_name="c", subcore_axis_name="s"),
    scratch_types=[
        pltpu.VMEM((_PE_ELEMS,), jnp.float32),
        pltpu.VMEM((_CH_ELEMS,), jnp.float32),
    ],
)(_sc_body)


def kernel(x, pe_weight):
    B, S, D = x.shape
    out = _sc_kernel(x.reshape(-1), pe_weight.reshape(-1))
    return out.reshape(B, S, D)


# speedup vs baseline: 1.5663x; 1.5663x over previous
import functools

import jax
import jax.numpy as jnp
from jax import lax
from jax.experimental import pallas as pl
from jax.experimental.pallas import tpu as pltpu
from jax.experimental.pallas import tpu_sc as plsc

_NC = 2
_NS = 16
_NW = _NC * _NS
_L = 16
_D = 1024
_BATCH = 4
_SEQ = 2048
_ROWS_W = _SEQ // _NW
_CH = 16
_CH_EL = _CH * _D
_PE_EL = _ROWS_W * _D
_CHUNKS_PER_BATCH = _ROWS_W // _CH
_T = _BATCH * _CHUNKS_PER_BATCH


def _sc_body(x_hbm, pe_hbm, o_hbm, pe_v, x_bufs,
             in_sem0, in_sem1, out_sem0, out_sem1):
    cid = lax.axis_index("c")
    sid = lax.axis_index("s")
    wid = sid * _NC + cid
    pe_base = wid * _PE_EL
    pltpu.sync_copy(pe_hbm.at[pl.ds(pe_base, _PE_EL)], pe_v)

    in_sems = (in_sem0, in_sem1)
    out_sems = (out_sem0, out_sem1)

    def row0(t):
        b = t // _CHUNKS_PER_BATCH
        c = t % _CHUNKS_PER_BATCH
        return b * (_SEQ * _D) + pe_base + c * _CH_EL

    in_d = [None, None]
    out_d = [None, None]

    def start_in(t, s):
        in_d[s] = pltpu.async_copy(
            x_hbm.at[pl.ds(row0(t), _CH_EL)], x_bufs.at[s], in_sems[s])

    def compute(s, t):
        c = t % _CHUNKS_PER_BATCH
        pe_off = c * _CH_EL

        @plsc.parallel_loop(0, _CH_EL // _L, unroll=8)
        def _(i):
            v = pe_v[pl.ds(pe_off + i * _L, _L)]
            plsc.addupdate(x_bufs.at[s, pl.ds(i * _L, _L)], v)

    start_in(0, 0)
    for t in range(_T):
        s = t % 2
        if t + 1 < _T:
            if t >= 1:
                out_d[1 - s].wait()
            start_in(t + 1, 1 - s)
        in_d[s].wait()
        compute(s, t)
        out_d[s] = pltpu.async_copy(
            x_bufs.at[s], o_hbm.at[pl.ds(row0(t), _CH_EL)], out_sems[s])
    out_d[0].wait()
    out_d[1].wait()


_sc_kernel = functools.partial(
    pl.kernel,
    out_type=jax.ShapeDtypeStruct((_BATCH * _SEQ * _D,), jnp.float32),
    mesh=plsc.VectorSubcoreMesh(core_axis_name="c", subcore_axis_name="s"),
    scratch_types=[
        pltpu.VMEM((_PE_EL,), jnp.float32),
        pltpu.VMEM((2, _CH_EL), jnp.float32),
        pltpu.SemaphoreType.DMA,
        pltpu.SemaphoreType.DMA,
        pltpu.SemaphoreType.DMA,
        pltpu.SemaphoreType.DMA,
    ],
)(_sc_body)


def kernel(x, pe_weight):
    B, S, D = x.shape
    out = _sc_kernel(x.reshape(-1), pe_weight.reshape(-1))
    return out.reshape(B, S, D)
